# baseline (speedup 1.0000x reference)
import functools

import jax
import jax.numpy as jnp
from jax import lax
from jax.experimental import pallas as pl
from jax.experimental.pallas import tpu as pltpu
from jax.experimental.pallas import tpu_sc as plsc

N = 10000
E = 320000
D = 128
DE = 16

NC = 2
NS = 16
L = 16
NW = NC * NS
EPW = E // NW
CH = 40
NCHUNK = EPW // CH
KD = D // L
N_PAD = 10240
RPT = N_PAD // NS
ZROWS = 128

f32 = jnp.float32
i32 = jnp.int32


def _dotT(a, b):
    return lax.dot_general(a, b, (((1,), (1,)), ((), ())),
                           preferred_element_type=f32)


BE = 4000


def _eprep_body(ea_ref, weg_ref, wer_ref, eg_ref, er_ref, easum_ref):
    i = pl.program_id(0)
    ea = ea_ref[...]
    eg_ref[...] = _dotT(ea, weg_ref[...])
    er_ref[...] = _dotT(ea, wer_ref[...])

    @pl.when(i == 0)
    def _():
        easum_ref[...] = jnp.zeros_like(easum_ref)

    s = jnp.sum(ea, axis=0, keepdims=True)
    easum_ref[...] += jnp.pad(s, ((0, 7), (0, D - DE)))


def _eprep(ea, weg, wer):
    return pl.pallas_call(
        _eprep_body,
        grid=(E // BE,),
        in_specs=[
            pl.BlockSpec((BE, DE), lambda i: (i, 0)),
            pl.BlockSpec((D, DE), lambda i: (0, 0)),
            pl.BlockSpec((D, DE), lambda i: (0, 0)),
        ],
        out_specs=[
            pl.BlockSpec((BE, D), lambda i: (i, 0)),
            pl.BlockSpec((BE, D), lambda i: (i, 0)),
            pl.BlockSpec((8, D), lambda i: (0, 0)),
        ],
        out_shape=[
            jax.ShapeDtypeStruct((E, D), f32),
            jax.ShapeDtypeStruct((E, D), f32),
            jax.ShapeDtypeStruct((8, D), f32),
        ],
    )(ea, weg, wer)


BN = 2000


def _nprep_body(easum_ref, x_ref, wl_ref, bl_ref, wr_ref, br_ref, att_ref,
                weg_ref, xl_ref, xr_ref, lout_ref, lden_ref):
    x = x_ref[...]
    xl = _dotT(x, wl_ref[...]) + bl_ref[...][None, :]
    xr = _dotT(x, wr_ref[...]) + br_ref[...][None, :]
    eam = easum_ref[0, :DE] * (1.0 / E)
    egl = jnp.dot(weg_ref[...], eam, preferred_element_type=f32)
    m = xl + xr + egl[None, :]
    m = jnp.where(m > 0, m, 0.2 * m)
    alpha = jnp.sum(m * att_ref[...][None, :], axis=1)
    ex = jnp.exp(alpha)
    xl_ref[...] = xl
    xr_ref[...] = xr
    lout_ref[...] = xl * ex[:, None]
    lden_ref[...] = jnp.pad(ex[:, None], ((0, 0), (0, 7)))


def _nprep(easum, x, wl, bl, wr, br, att, weg):
    return pl.pallas_call(
        _nprep_body,
        grid=(N // BN,),
        in_specs=[
            pl.BlockSpec((8, D), lambda i: (0, 0)),
            pl.BlockSpec((BN, D), lambda i: (i, 0)),
            pl.BlockSpec((D, D), lambda i: (0, 0)),
            pl.BlockSpec((D,), lambda i: (0,)),
            pl.BlockSpec((D, D), lambda i: (0, 0)),
            pl.BlockSpec((D,), lambda i: (0,)),
            pl.BlockSpec((D,), lambda i: (0,)),
            pl.BlockSpec((D, DE), lambda i: (0, 0)),
        ],
        out_specs=[
            pl.BlockSpec((BN, D), lambda i: (i, 0)),
            pl.BlockSpec((BN, D), lambda i: (i, 0)),
            pl.BlockSpec((BN, D), lambda i: (i, 0)),
            pl.BlockSpec((BN, 8), lambda i: (i, 0)),
        ],
        out_shape=[
            jax.ShapeDtypeStruct((N, D), f32),
            jax.ShapeDtypeStruct((N, D), f32),
            jax.ShapeDtypeStruct((N, D), f32),
            jax.ShapeDtypeStruct((N, 8), f32),
        ],
    )(easum, x, wl, bl, wr, br, att, weg)


def _sc_gat_body(xl_hbm, xr_hbm, eg_hbm, idx_hbm, att_hbm,
                 zout_hbm, zden_hbm,
                 out_hbm, den_hbm,
                 xv0, rv0, gv0, xv1, rv1, gv1, iv0, iv1, iv2, iv3,
                 dv0, dv1, attv,
                 sem0, sem1, ssem0, ssem1,
                 isem0, isem1, isem2, isem3, out_sh, den_sh):
    c = lax.axis_index("c")
    s = lax.axis_index("s")
    w = c * NS + s
    iota = lax.iota(i32, L)
    z16 = jnp.zeros((L,), f32)
    onehot = jnp.where(iota == 0, 1.0, 0.0)

    def _dv_row(i, carry):
        dv0[i] = z16
        dv1[i] = z16
        return carry
    lax.fori_loop(0, CH, _dv_row, 0)

    pltpu.sync_copy(zout_hbm, out_sh.at[pl.ds(s * RPT, RPT)])
    pltpu.sync_copy(zden_hbm, den_sh.at[pl.ds(s * RPT, RPT)])
    pltpu.sync_copy(att_hbm, attv)
    plsc.subcore_barrier()

    attk = [attv[0, pl.ds(k * L, L)] for k in range(KD)]
    slots = [(xv0, rv0, gv0, dv0, sem0, ssem0),
             (xv1, rv1, gv1, dv1, sem1, ssem1)]
    ivs = (iv0, iv1, iv2, iv3)
    isems = (isem0, isem1, isem2, isem3)

    def _issue_idx(j, q):
        pltpu.async_copy(idx_hbm.at[w, j], ivs[q], isems[q])

    def _wait_idx(j, q):
        pltpu.make_async_copy(idx_hbm.at[w, j], ivs[q], isems[q]).wait()

    def _prefetch(j, slot, q, drain_scatter):
        xv, rv, gv, dv, sem, ssem = slots[slot]
        ivp = ivs[(q + 2) % 4]
        if drain_scatter:
            pltpu.make_async_copy(xv, out_sh.at[ivp.at[1]], ssem).wait()
            pltpu.make_async_copy(dv, den_sh.at[ivp.at[1]], ssem).wait()
        iv = ivs[q]
        _wait_idx(j, q)
        pltpu.async_copy(xl_hbm.at[iv.at[0]], xv, sem)
        pltpu.async_copy(xr_hbm.at[iv.at[1]], rv, sem)
        pltpu.async_copy(eg_hbm.at[pl.ds((w * NCHUNK + j) * CH, CH)], gv, sem)

    def _process(j, slot, q):
        xv, rv, gv, dv, sem, ssem = slots[slot]
        iv = ivs[q]
        pltpu.make_async_copy(xl_hbm.at[iv.at[0]], xv, sem).wait()
        pltpu.make_async_copy(xr_hbm.at[iv.at[1]], rv, sem).wait()
        pltpu.make_async_copy(eg_hbm.at[pl.ds(0, CH)], gv, sem).wait()

        @plsc.parallel_loop(0, CH, unroll=3)
        def _edge(e):
            xs = []
            t = z16
            for k in range(KD):
                ds = pl.ds(k * L, L)
                x = xv[e, ds]
                m = x + rv[e, ds] + gv[e, ds]
                m = jnp.where(m > 0, m, 0.2 * m)
                t = t + m * attk[k]
                xs.append(x)
            tot = jnp.sum(t)
            exv = jnp.exp(jnp.full((L,), tot, f32))
            for k in range(KD):
                xv[e, pl.ds(k * L, L)] = xs[k] * exv
            dv[e] = onehot * exv
        pltpu.async_copy(xv, out_sh.at[iv.at[1]], ssem, add=True)
        pltpu.async_copy(dv, den_sh.at[iv.at[1]], ssem, add=True)

        @pl.when(j + 2 < NCHUNK)
        def _():
            _prefetch(j + 2, slot, (q + 2) % 4, True)

        @pl.when(j + 4 < NCHUNK)
        def _():
            _issue_idx(j + 4, q)

        @pl.when(j + 2 >= NCHUNK)
        def _():
            pltpu.make_async_copy(xv, out_sh.at[iv.at[1]], ssem).wait()
            pltpu.make_async_copy(dv, den_sh.at[iv.at[1]], ssem).wait()

    for q0 in range(4):
        _issue_idx(q0, q0)
    _prefetch(0, 0, 0, False)
    _prefetch(1, 1, 1, False)

    def _quad(i, carry):
        _process(4 * i, 0, 0)
        _process(4 * i + 1, 1, 1)
        _process(4 * i + 2, 0, 2)
        _process(4 * i + 3, 1, 3)
        return carry

    lax.fori_loop(0, NCHUNK // 4, _quad, 0)
    _process(NCHUNK - 2, 0, 0)
    _process(NCHUNK - 1, 1, 1)
    plsc.subcore_barrier()
    pltpu.sync_copy(out_sh.at[pl.ds(s * RPT, RPT)],
                    out_hbm.at[c, pl.ds(s * RPT, RPT)])
    pltpu.sync_copy(den_sh.at[pl.ds(s * RPT, RPT)],
                    den_hbm.at[c, pl.ds(s * RPT, RPT)])


def _gat_edges(xl, xr, eg, idx, att):
    mesh = plsc.VectorSubcoreMesh(core_axis_name="c", subcore_axis_name="s",
                                  num_cores=NC, num_subcores=NS)
    return pl.kernel(
        _sc_gat_body,
        out_type=[
            jax.ShapeDtypeStruct((NC, N_PAD, D), f32),
            jax.ShapeDtypeStruct((NC, N_PAD, DE), f32),
        ],
        mesh=mesh,
        scratch_types=[
            pltpu.VMEM((CH, D), f32),
            pltpu.VMEM((CH, D), f32),
            pltpu.VMEM((CH, D), f32),
            pltpu.VMEM((CH, D), f32),
            pltpu.VMEM((CH, D), f32),
            pltpu.VMEM((CH, D), f32),
            pltpu.VMEM((2, CH), i32),
            pltpu.VMEM((2, CH), i32),
            pltpu.VMEM((2, CH), i32),
            pltpu.VMEM((2, CH), i32),
            pltpu.VMEM((CH, DE), f32),
            pltpu.VMEM((CH, DE), f32),
            pltpu.VMEM((1, D), f32),
            pltpu.SemaphoreType.DMA,
            pltpu.SemaphoreType.DMA,
            pltpu.SemaphoreType.DMA,
            pltpu.SemaphoreType.DMA,
            pltpu.SemaphoreType.DMA,
            pltpu.SemaphoreType.DMA,
            pltpu.SemaphoreType.DMA,
            pltpu.SemaphoreType.DMA,
            pltpu.VMEM_SHARED((N_PAD, D), f32),
            pltpu.VMEM_SHARED((N_PAD, DE), f32),
        ],
        compiler_params=pltpu.CompilerParams(needs_layout_passes=False,
                                             use_tc_tiling_on_sc=False),
    )(xl, xr, eg, idx, att.reshape(1, D),
      jnp.zeros((RPT, D), f32), jnp.zeros((RPT, DE), f32))


def _comb_body(outsc_ref, densc_ref, lout_ref, lden_ref, gbias_ref,
               wk_ref, bk_ref, wq_ref, bq_ref, wv_ref, bv_ref, wskip_ref,
               k_ref, q_ref, v_ref, skip_ref):
    num = outsc_ref[0] + outsc_ref[1] + lout_ref[...]
    den = (jnp.sum(densc_ref[0], axis=-1) + jnp.sum(densc_ref[1], axis=-1)
           + jnp.sum(lden_ref[...], axis=-1) + 1e-16)
    h = num / den[:, None] + gbias_ref[...][None, :]
    k_ref[...] = _dotT(h, wk_ref[...]) + bk_ref[...][None, :]
    q_ref[...] = _dotT(h, wq_ref[...]) + bq_ref[...][None, :]
    v_ref[...] = _dotT(h, wv_ref[...]) + bv_ref[...][None, :]
    skip_ref[...] = _dotT(h, wskip_ref[...])


def _combine(outsc, densc, lout, lden, gbias, wk, bk, wq, bq, wv, bv, wskip):
    full = lambda shp: pl.BlockSpec(shp, lambda i: tuple(0 for _ in shp))
    return pl.pallas_call(
        _comb_body,
        grid=(N // BN,),
        in_specs=[
            pl.BlockSpec((NC, BN, D), lambda i: (0, i, 0)),
            pl.BlockSpec((NC, BN, DE), lambda i: (0, i, 0)),
            pl.BlockSpec((BN, D), lambda i: (i, 0)),
            pl.BlockSpec((BN, 8), lambda i: (i, 0)),
            pl.BlockSpec((D,), lambda i: (0,)),
            full((D, D)), pl.BlockSpec((D,), lambda i: (0,)),
            full((D, D)), pl.BlockSpec((D,), lambda i: (0,)),
            full((D, D)), pl.BlockSpec((D,), lambda i: (0,)),
            full((D, D)),
        ],
        out_specs=[pl.BlockSpec((BN, D), lambda i: (i, 0))] * 4,
        out_shape=[jax.ShapeDtypeStruct((N, D), f32)] * 4,
    )(outsc, densc, lout, lden, gbias, wk, bk, wq, bq, wv, bv, wskip)


def _sc_res_body(k_hbm, q_hbm, v_hbm, er_hbm, idx_hbm, zout_hbm,
                 out_hbm,
                 kv0, qv0, vv0, ev0, kv1, qv1, vv1, ev1,
                 iv0, iv1, iv2, iv3,
                 sem0, sem1, ssem0, ssem1,
                 isem0, isem1, isem2, isem3, out_sh):
    c = lax.axis_index("c")
    s = lax.axis_index("s")
    w = c * NS + s

    pltpu.sync_copy(zout_hbm, out_sh.at[pl.ds(s * RPT, RPT)])
    plsc.subcore_barrier()

    slots = [(kv0, qv0, vv0, ev0, sem0, ssem0),
             (kv1, qv1, vv1, ev1, sem1, ssem1)]
    ivs = (iv0, iv1, iv2, iv3)
    isems = (isem0, isem1, isem2, isem3)

    def _issue_idx(j, q):
        pltpu.async_copy(idx_hbm.at[w, j], ivs[q], isems[q])

    def _wait_idx(j, q):
        pltpu.make_async_copy(idx_hbm.at[w, j], ivs[q], isems[q]).wait()

    def _prefetch(j, slot, q, drain_scatter):
        kv, qv, vv, ev, sem, ssem = slots[slot]
        ivp = ivs[(q + 2) % 4]
        if drain_scatter:
            pltpu.make_async_copy(vv, out_sh.at[ivp.at[1]], ssem).wait()
        iv = ivs[q]
        _wait_idx(j, q)
        pltpu.async_copy(k_hbm.at[iv.at[1]], kv, sem)
        pltpu.async_copy(q_hbm.at[iv.at[0]], qv, sem)
        pltpu.async_copy(v_hbm.at[iv.at[0]], vv, sem)
        pltpu.async_copy(er_hbm.at[pl.ds((w * NCHUNK + j) * CH, CH)], ev, sem)

    def _process(j, slot, q):
        kv, qv, vv, ev, sem, ssem = slots[slot]
        iv = ivs[q]
        pltpu.make_async_copy(k_hbm.at[iv.at[1]], kv, sem).wait()
        pltpu.make_async_copy(q_hbm.at[iv.at[0]], qv, sem).wait()
        pltpu.make_async_copy(v_hbm.at[iv.at[0]], vv, sem).wait()
        pltpu.make_async_copy(er_hbm.at[pl.ds(0, CH)], ev, sem).wait()

        @plsc.parallel_loop(0, CH, unroll=3)
        def _edge(e):
            for k in range(KD):
                ds = pl.ds(k * L, L)
                z = kv[e, ds] + ev[e, ds] + qv[e, ds]
                eta = 1.0 / (1.0 + jnp.exp(-z))
                vv[e, ds] = eta * vv[e, ds]
        pltpu.async_copy(vv, out_sh.at[iv.at[1]], ssem, add=True)

        @pl.when(j + 2 < NCHUNK)
        def _():
            _prefetch(j + 2, slot, (q + 2) % 4, True)

        @pl.when(j + 4 < NCHUNK)
        def _():
            _issue_idx(j + 4, q)

        @pl.when(j + 2 >= NCHUNK)
        def _():
            pltpu.make_async_copy(vv, out_sh.at[iv.at[1]], ssem).wait()

    for q0 in range(4):
        _issue_idx(q0, q0)
    _prefetch(0, 0, 0, False)
    _prefetch(1, 1, 1, False)

    def _quad(i, carry):
        _process(4 * i, 0, 0)
        _process(4 * i + 1, 1, 1)
        _process(4 * i + 2, 0, 2)
        _process(4 * i + 3, 1, 3)
        return carry

    lax.fori_loop(0, NCHUNK // 4, _quad, 0)
    _process(NCHUNK - 2, 0, 0)
    _process(NCHUNK - 1, 1, 1)
    plsc.subcore_barrier()
    pltpu.sync_copy(out_sh.at[pl.ds(s * RPT, RPT)],
                    out_hbm.at[c, pl.ds(s * RPT, RPT)])


def _res_edges(kk, qq, vv, er, idx):
    mesh = plsc.VectorSubcoreMesh(core_axis_name="c", subcore_axis_name="s",
                                  num_cores=NC, num_subcores=NS)
    return pl.kernel(
        _sc_res_body,
        out_type=jax.ShapeDtypeStruct((NC, N_PAD, D), f32),
        mesh=mesh,
        scratch_types=[
            pltpu.VMEM((CH, D), f32),
            pltpu.VMEM((CH, D), f32),
            pltpu.VMEM((CH, D), f32),
            pltpu.VMEM((CH, D), f32),
            pltpu.VMEM((CH, D), f32),
            pltpu.VMEM((CH, D), f32),
            pltpu.VMEM((CH, D), f32),
            pltpu.VMEM((CH, D), f32),
            pltpu.VMEM((2, CH), i32),
            pltpu.VMEM((2, CH), i32),
            pltpu.VMEM((2, CH), i32),
            pltpu.VMEM((2, CH), i32),
            pltpu.SemaphoreType.DMA,
            pltpu.SemaphoreType.DMA,
            pltpu.SemaphoreType.DMA,
            pltpu.SemaphoreType.DMA,
            pltpu.SemaphoreType.DMA,
            pltpu.SemaphoreType.DMA,
            pltpu.SemaphoreType.DMA,
            pltpu.SemaphoreType.DMA,
            pltpu.VMEM_SHARED((N_PAD, D), f32),
        ],
        compiler_params=pltpu.CompilerParams(needs_layout_passes=False,
                                             use_tc_tiling_on_sc=False),
    )(kk, qq, vv, er, idx, jnp.zeros((RPT, D), f32))


def _k5a_body(out2_ref, skip_ref, rbias_ref, h2_ref, stats_ref):
    i = pl.program_id(0)
    h2 = out2_ref[0] + out2_ref[1] + skip_ref[...] + rbias_ref[...][None, :]
    h2 = jnp.where(h2 > 0, h2, 0.01 * h2)
    h2_ref[...] = h2

    @pl.when(i == 0)
    def _():
        stats_ref[...] = jnp.zeros_like(stats_ref)

    ssum = jnp.sum(h2, axis=0, keepdims=True)
    ssq = jnp.sum(h2 * h2, axis=0, keepdims=True)
    stats_ref[...] += jnp.concatenate(
        [ssum, ssq, jnp.zeros((6, D), f32)], axis=0)


def _k5a(out2, skip, rbias):
    return pl.pallas_call(
        _k5a_body,
        grid=(N // BN,),
        in_specs=[
            pl.BlockSpec((NC, BN, D), lambda i: (0, i, 0)),
            pl.BlockSpec((BN, D), lambda i: (i, 0)),
            pl.BlockSpec((D,), lambda i: (0,)),
        ],
        out_specs=[
            pl.BlockSpec((BN, D), lambda i: (i, 0)),
            pl.BlockSpec((8, D), lambda i: (0, 0)),
        ],
        out_shape=[
            jax.ShapeDtypeStruct((N, D), f32),
            jax.ShapeDtypeStruct((8, D), f32),
        ],
    )(out2, skip, rbias)


def _k5b_body(h2_ref, stats_ref, gamma_ref, beta_ref, out_ref):
    mu = stats_ref[0, :] * (1.0 / N)
    var = stats_ref[1, :] * (1.0 / N) - mu * mu
    inv = 1.0 / jnp.sqrt(var + 1e-5)
    out_ref[...] = ((h2_ref[...] - mu[None, :]) * (inv * gamma_ref[...])[None, :]
                    + beta_ref[...][None, :])


def _k5b(h2, stats, gamma, beta):
    return pl.pallas_call(
        _k5b_body,
        grid=(N // BN,),
        in_specs=[
            pl.BlockSpec((BN, D), lambda i: (i, 0)),
            pl.BlockSpec((8, D), lambda i: (0, 0)),
            pl.BlockSpec((D,), lambda i: (0,)),
            pl.BlockSpec((D,), lambda i: (0,)),
        ],
        out_specs=pl.BlockSpec((BN, D), lambda i: (i, 0)),
        out_shape=jax.ShapeDtypeStruct((N, D), f32),
    )(h2, stats, gamma, beta)


@jax.jit
def kernel(x, edge_index, edge_attr, gat_Wl, gat_bl, gat_Wr, gat_br, gat_att,
           gat_We, gat_bias, res_Wk, res_bk, res_Wq, res_bq, res_Wv, res_bv,
           res_We, res_Wskip, res_bias, bn_gamma, bn_beta):
    idx = jnp.stack([edge_index[0].astype(i32).reshape(NW, NCHUNK, CH),
                     edge_index[1].astype(i32).reshape(NW, NCHUNK, CH)],
                    axis=2)

    eg, er, easum = _eprep(edge_attr, gat_We, res_We)
    xl, xr, lout, lden = _nprep(easum, x, gat_Wl, gat_bl, gat_Wr, gat_br,
                                gat_att, gat_We)
    outsc, densc = _gat_edges(xl, xr, eg, idx, gat_att)
    kk, qq, vv, skip = _combine(outsc, densc, lout, lden, gat_bias,
                                res_Wk, res_bk, res_Wq, res_bq,
                                res_Wv, res_bv, res_Wskip)
    out2 = _res_edges(kk, qq, vv, er, idx)
    h2, stats = _k5a(out2, skip, res_bias)
    return _k5b(h2, stats, bn_gamma, bn_beta)


# speedup vs baseline: 1.1414x; 1.1414x over previous
import functools

import jax
import jax.numpy as jnp
from jax import lax
from jax.experimental import pallas as pl
from jax.experimental.pallas import tpu as pltpu
from jax.experimental.pallas import tpu_sc as plsc

N = 10000
E = 320000
D = 128
DE = 16

NC = 2
NS = 16
L = 16
NW = NC * NS
EPW = E // NW
CH = 40
NCHUNK = EPW // CH
KD = D // L
N_PAD = 10240
RPT = N_PAD // NS
ZROWS = 128

f32 = jnp.float32
i32 = jnp.int32


def _dotT(a, b):
    return lax.dot_general(a, b, (((1,), (1,)), ((), ())),
                           preferred_element_type=f32)


BE = 4000


def _eprep_body(ea_ref, weg_ref, wer_ref, eg_ref, er_ref, easum_ref):
    i = pl.program_id(0)
    ea = ea_ref[...]
    eg_ref[...] = _dotT(ea, weg_ref[...])
    er_ref[...] = _dotT(ea, wer_ref[...])

    @pl.when(i == 0)
    def _():
        easum_ref[...] = jnp.zeros_like(easum_ref)

    s = jnp.sum(ea, axis=0, keepdims=True)
    easum_ref[...] += jnp.pad(s, ((0, 7), (0, D - DE)))


def _eprep(ea, weg, wer):
    return pl.pallas_call(
        _eprep_body,
        grid=(E // BE,),
        in_specs=[
            pl.BlockSpec((BE, DE), lambda i: (i, 0)),
            pl.BlockSpec((D, DE), lambda i: (0, 0)),
            pl.BlockSpec((D, DE), lambda i: (0, 0)),
        ],
        out_specs=[
            pl.BlockSpec((BE, D), lambda i: (i, 0)),
            pl.BlockSpec((BE, D), lambda i: (i, 0)),
            pl.BlockSpec((8, D), lambda i: (0, 0)),
        ],
        out_shape=[
            jax.ShapeDtypeStruct((E, D), f32),
            jax.ShapeDtypeStruct((E, D), f32),
            jax.ShapeDtypeStruct((8, D), f32),
        ],
    )(ea, weg, wer)


BN = 2000


def _nprep_body(easum_ref, x_ref, wl_ref, bl_ref, wr_ref, br_ref, att_ref,
                weg_ref, xl_ref, xr_ref, lout_ref, lden_ref):
    x = x_ref[...]
    xl = _dotT(x, wl_ref[...]) + bl_ref[...][None, :]
    xr = _dotT(x, wr_ref[...]) + br_ref[...][None, :]
    eam = easum_ref[0, :DE] * (1.0 / E)
    egl = jnp.dot(weg_ref[...], eam, preferred_element_type=f32)
    m = xl + xr + egl[None, :]
    m = jnp.where(m > 0, m, 0.2 * m)
    alpha = jnp.sum(m * att_ref[...][None, :], axis=1)
    ex = jnp.exp(alpha)
    xl_ref[...] = xl
    xr_ref[...] = xr
    lout_ref[...] = xl * ex[:, None]
    lden_ref[...] = jnp.pad(ex[:, None], ((0, 0), (0, 7)))


def _nprep(easum, x, wl, bl, wr, br, att, weg):
    return pl.pallas_call(
        _nprep_body,
        grid=(N // BN,),
        in_specs=[
            pl.BlockSpec((8, D), lambda i: (0, 0)),
            pl.BlockSpec((BN, D), lambda i: (i, 0)),
            pl.BlockSpec((D, D), lambda i: (0, 0)),
            pl.BlockSpec((D,), lambda i: (0,)),
            pl.BlockSpec((D, D), lambda i: (0, 0)),
            pl.BlockSpec((D,), lambda i: (0,)),
            pl.BlockSpec((D,), lambda i: (0,)),
            pl.BlockSpec((D, DE), lambda i: (0, 0)),
        ],
        out_specs=[
            pl.BlockSpec((BN, D), lambda i: (i, 0)),
            pl.BlockSpec((BN, D), lambda i: (i, 0)),
            pl.BlockSpec((BN, D), lambda i: (i, 0)),
            pl.BlockSpec((BN, 8), lambda i: (i, 0)),
        ],
        out_shape=[
            jax.ShapeDtypeStruct((N, D), f32),
            jax.ShapeDtypeStruct((N, D), f32),
            jax.ShapeDtypeStruct((N, D), f32),
            jax.ShapeDtypeStruct((N, 8), f32),
        ],
    )(easum, x, wl, bl, wr, br, att, weg)


def _sc_gat_body(xl_hbm, xr_hbm, eg_hbm, idx_hbm, att_hbm,
                 zout_hbm, zden_hbm,
                 out_hbm, den_hbm,
                 xv0, rv0, gv0, xv1, rv1, gv1, iv0, iv1, iv2, iv3,
                 dv0, dv1, attv,
                 sem0, sem1, ssem0, ssem1,
                 isem0, isem1, isem2, isem3, out_sh, den_sh):
    c = lax.axis_index("c")
    s = lax.axis_index("s")
    w = c * NS + s
    iota = lax.iota(i32, L)
    z16 = jnp.zeros((L,), f32)
    onehot = jnp.where(iota == 0, 1.0, 0.0)

    def _dv_row(i, carry):
        dv0[i] = z16
        dv1[i] = z16
        return carry
    lax.fori_loop(0, CH, _dv_row, 0)

    pltpu.sync_copy(zout_hbm, out_sh.at[pl.ds(s * RPT, RPT)])
    pltpu.sync_copy(zden_hbm, den_sh.at[pl.ds(s * RPT, RPT)])
    pltpu.sync_copy(att_hbm, attv)
    plsc.subcore_barrier()

    attk = [attv[0, pl.ds(k * L, L)] for k in range(KD)]
    slots = [(xv0, rv0, gv0, dv0, sem0, ssem0),
             (xv1, rv1, gv1, dv1, sem1, ssem1)]
    ivs = (iv0, iv1, iv2, iv3)
    isems = (isem0, isem1, isem2, isem3)

    def _issue_idx(j, q):
        pltpu.async_copy(idx_hbm.at[w, j], ivs[q], isems[q])

    def _wait_idx(j, q):
        pltpu.make_async_copy(idx_hbm.at[w, j], ivs[q], isems[q]).wait()

    def _prefetch(j, slot, q, drain_scatter):
        xv, rv, gv, dv, sem, ssem = slots[slot]
        ivp = ivs[(q + 2) % 4]
        if drain_scatter:
            pltpu.make_async_copy(xv, out_sh.at[ivp.at[1]], ssem).wait()
            pltpu.make_async_copy(dv, den_sh.at[ivp.at[1]], ssem).wait()
        iv = ivs[q]
        _wait_idx(j, q)
        pltpu.async_copy(xl_hbm.at[iv.at[0]], xv, sem)
        pltpu.async_copy(xr_hbm.at[iv.at[1]], rv, sem)
        pltpu.async_copy(eg_hbm.at[pl.ds((w * NCHUNK + j) * CH, CH)], gv, sem)

    def _process(j, slot, q):
        xv, rv, gv, dv, sem, ssem = slots[slot]
        iv = ivs[q]
        pltpu.make_async_copy(xl_hbm.at[iv.at[0]], xv, sem).wait()
        pltpu.make_async_copy(xr_hbm.at[iv.at[1]], rv, sem).wait()
        pltpu.make_async_copy(eg_hbm.at[pl.ds(0, CH)], gv, sem).wait()

        @plsc.parallel_loop(0, CH, unroll=1)
        def _edge(e):
            xs = []
            t = z16
            for k in range(KD):
                ds = pl.ds(k * L, L)
                x = xv[e, ds]
                m = x + rv[e, ds] + gv[e, ds]
                m = jnp.where(m > 0, m, 0.2 * m)
                t = t + m * attk[k]
                xs.append(x)
            tot = jnp.sum(t)
            exv = jnp.exp(jnp.full((L,), tot, f32))
            for k in range(KD):
                xv[e, pl.ds(k * L, L)] = xs[k] * exv
            dv[e] = onehot * exv
        pltpu.async_copy(xv, out_sh.at[iv.at[1]], ssem, add=True)
        pltpu.async_copy(dv, den_sh.at[iv.at[1]], ssem, add=True)

        @pl.when(j + 2 < NCHUNK)
        def _():
            _prefetch(j + 2, slot, (q + 2) % 4, True)

        @pl.when(j + 4 < NCHUNK)
        def _():
            _issue_idx(j + 4, q)

        @pl.when(j + 2 >= NCHUNK)
        def _():
            pltpu.make_async_copy(xv, out_sh.at[iv.at[1]], ssem).wait()
            pltpu.make_async_copy(dv, den_sh.at[iv.at[1]], ssem).wait()

    for q0 in range(4):
        _issue_idx(q0, q0)
    _prefetch(0, 0, 0, False)
    _prefetch(1, 1, 1, False)

    def _quad(i, carry):
        _process(4 * i, 0, 0)
        _process(4 * i + 1, 1, 1)
        _process(4 * i + 2, 0, 2)
        _process(4 * i + 3, 1, 3)
        return carry

    lax.fori_loop(0, NCHUNK // 4, _quad, 0)
    _process(NCHUNK - 2, 0, 0)
    _process(NCHUNK - 1, 1, 1)
    plsc.subcore_barrier()
    pltpu.sync_copy(out_sh.at[pl.ds(s * RPT, RPT)],
                    out_hbm.at[c, pl.ds(s * RPT, RPT)])
    pltpu.sync_copy(den_sh.at[pl.ds(s * RPT, RPT)],
                    den_hbm.at[c, pl.ds(s * RPT, RPT)])


def _gat_edges(xl, xr, eg, idx, att):
    mesh = plsc.VectorSubcoreMesh(core_axis_name="c", subcore_axis_name="s",
                                  num_cores=NC, num_subcores=NS)
    return pl.kernel(
        _sc_gat_body,
        out_type=[
            jax.ShapeDtypeStruct((NC, N_PAD, D), f32),
            jax.ShapeDtypeStruct((NC, N_PAD, DE), f32),
        ],
        mesh=mesh,
        scratch_types=[
            pltpu.VMEM((CH, D), f32),
            pltpu.VMEM((CH, D), f32),
            pltpu.VMEM((CH, D), f32),
            pltpu.VMEM((CH, D), f32),
            pltpu.VMEM((CH, D), f32),
            pltpu.VMEM((CH, D), f32),
            pltpu.VMEM((2, CH), i32),
            pltpu.VMEM((2, CH), i32),
            pltpu.VMEM((2, CH), i32),
            pltpu.VMEM((2, CH), i32),
            pltpu.VMEM((CH, DE), f32),
            pltpu.VMEM((CH, DE), f32),
            pltpu.VMEM((1, D), f32),
            pltpu.SemaphoreType.DMA,
            pltpu.SemaphoreType.DMA,
            pltpu.SemaphoreType.DMA,
            pltpu.SemaphoreType.DMA,
            pltpu.SemaphoreType.DMA,
            pltpu.SemaphoreType.DMA,
            pltpu.SemaphoreType.DMA,
            pltpu.SemaphoreType.DMA,
            pltpu.VMEM_SHARED((N_PAD, D), f32),
            pltpu.VMEM_SHARED((N_PAD, DE), f32),
        ],
        compiler_params=pltpu.CompilerParams(needs_layout_passes=False,
                                             use_tc_tiling_on_sc=False),
    )(xl, xr, eg, idx, att.reshape(1, D),
      jnp.zeros((RPT, D), f32), jnp.zeros((RPT, DE), f32))


def _comb_body(outsc_ref, densc_ref, lout_ref, lden_ref, gbias_ref,
               wk_ref, bk_ref, wq_ref, bq_ref, wv_ref, bv_ref, wskip_ref,
               k_ref, q_ref, v_ref, skip_ref):
    num = outsc_ref[0] + outsc_ref[1] + lout_ref[...]
    den = (jnp.sum(densc_ref[0], axis=-1) + jnp.sum(densc_ref[1], axis=-1)
           + jnp.sum(lden_ref[...], axis=-1) + 1e-16)
    h = num / den[:, None] + gbias_ref[...][None, :]
    k_ref[...] = _dotT(h, wk_ref[...]) + bk_ref[...][None, :]
    q_ref[...] = _dotT(h, wq_ref[...]) + bq_ref[...][None, :]
    v_ref[...] = _dotT(h, wv_ref[...]) + bv_ref[...][None, :]
    skip_ref[...] = _dotT(h, wskip_ref[...])


def _combine(outsc, densc, lout, lden, gbias, wk, bk, wq, bq, wv, bv, wskip):
    full = lambda shp: pl.BlockSpec(shp, lambda i: tuple(0 for _ in shp))
    return pl.pallas_call(
        _comb_body,
        grid=(N // BN,),
        in_specs=[
            pl.BlockSpec((NC, BN, D), lambda i: (0, i, 0)),
            pl.BlockSpec((NC, BN, DE), lambda i: (0, i, 0)),
            pl.BlockSpec((BN, D), lambda i: (i, 0)),
            pl.BlockSpec((BN, 8), lambda i: (i, 0)),
            pl.BlockSpec((D,), lambda i: (0,)),
            full((D, D)), pl.BlockSpec((D,), lambda i: (0,)),
            full((D, D)), pl.BlockSpec((D,), lambda i: (0,)),
            full((D, D)), pl.BlockSpec((D,), lambda i: (0,)),
            full((D, D)),
        ],
        out_specs=[pl.BlockSpec((BN, D), lambda i: (i, 0))] * 4,
        out_shape=[jax.ShapeDtypeStruct((N, D), f32)] * 4,
    )(outsc, densc, lout, lden, gbias, wk, bk, wq, bq, wv, bv, wskip)


def _sc_res_body(k_hbm, q_hbm, v_hbm, er_hbm, idx_hbm, zout_hbm,
                 out_hbm,
                 kv0, qv0, vv0, ev0, kv1, qv1, vv1, ev1,
                 iv0, iv1, iv2, iv3,
                 sem0, sem1, ssem0, ssem1,
                 isem0, isem1, isem2, isem3, out_sh):
    c = lax.axis_index("c")
    s = lax.axis_index("s")
    w = c * NS + s

    pltpu.sync_copy(zout_hbm, out_sh.at[pl.ds(s * RPT, RPT)])
    plsc.subcore_barrier()

    slots = [(kv0, qv0, vv0, ev0, sem0, ssem0),
             (kv1, qv1, vv1, ev1, sem1, ssem1)]
    ivs = (iv0, iv1, iv2, iv3)
    isems = (isem0, isem1, isem2, isem3)

    def _issue_idx(j, q):
        pltpu.async_copy(idx_hbm.at[w, j], ivs[q], isems[q])

    def _wait_idx(j, q):
        pltpu.make_async_copy(idx_hbm.at[w, j], ivs[q], isems[q]).wait()

    def _prefetch(j, slot, q, drain_scatter):
        kv, qv, vv, ev, sem, ssem = slots[slot]
        ivp = ivs[(q + 2) % 4]
        if drain_scatter:
            pltpu.make_async_copy(vv, out_sh.at[ivp.at[1]], ssem).wait()
        iv = ivs[q]
        _wait_idx(j, q)
        pltpu.async_copy(k_hbm.at[iv.at[1]], kv, sem)
        pltpu.async_copy(q_hbm.at[iv.at[0]], qv, sem)
        pltpu.async_copy(v_hbm.at[iv.at[0]], vv, sem)
        pltpu.async_copy(er_hbm.at[pl.ds((w * NCHUNK + j) * CH, CH)], ev, sem)

    def _process(j, slot, q):
        kv, qv, vv, ev, sem, ssem = slots[slot]
        iv = ivs[q]
        pltpu.make_async_copy(k_hbm.at[iv.at[1]], kv, sem).wait()
        pltpu.make_async_copy(q_hbm.at[iv.at[0]], qv, sem).wait()
        pltpu.make_async_copy(v_hbm.at[iv.at[0]], vv, sem).wait()
        pltpu.make_async_copy(er_hbm.at[pl.ds(0, CH)], ev, sem).wait()

        @plsc.parallel_loop(0, CH, unroll=1)
        def _edge(e):
            for k in range(KD):
                ds = pl.ds(k * L, L)
                z = kv[e, ds] + ev[e, ds] + qv[e, ds]
                eta = 1.0 / (1.0 + jnp.exp(-z))
                vv[e, ds] = eta * vv[e, ds]
        pltpu.async_copy(vv, out_sh.at[iv.at[1]], ssem, add=True)

        @pl.when(j + 2 < NCHUNK)
        def _():
            _prefetch(j + 2, slot, (q + 2) % 4, True)

        @pl.when(j + 4 < NCHUNK)
        def _():
            _issue_idx(j + 4, q)

        @pl.when(j + 2 >= NCHUNK)
        def _():
            pltpu.make_async_copy(vv, out_sh.at[iv.at[1]], ssem).wait()

    for q0 in range(4):
        _issue_idx(q0, q0)
    _prefetch(0, 0, 0, False)
    _prefetch(1, 1, 1, False)

    def _quad(i, carry):
        _process(4 * i, 0, 0)
        _process(4 * i + 1, 1, 1)
        _process(4 * i + 2, 0, 2)
        _process(4 * i + 3, 1, 3)
        return carry

    lax.fori_loop(0, NCHUNK // 4, _quad, 0)
    _process(NCHUNK - 2, 0, 0)
    _process(NCHUNK - 1, 1, 1)
    plsc.subcore_barrier()
    pltpu.sync_copy(out_sh.at[pl.ds(s * RPT, RPT)],
                    out_hbm.at[c, pl.ds(s * RPT, RPT)])


def _res_edges(kk, qq, vv, er, idx):
    mesh = plsc.VectorSubcoreMesh(core_axis_name="c", subcore_axis_name="s",
                                  num_cores=NC, num_subcores=NS)
    return pl.kernel(
        _sc_res_body,
        out_type=jax.ShapeDtypeStruct((NC, N_PAD, D), f32),
        mesh=mesh,
        scratch_types=[
            pltpu.VMEM((CH, D), f32),
            pltpu.VMEM((CH, D), f32),
            pltpu.VMEM((CH, D), f32),
            pltpu.VMEM((CH, D), f32),
            pltpu.VMEM((CH, D), f32),
            pltpu.VMEM((CH, D), f32),
            pltpu.VMEM((CH, D), f32),
            pltpu.VMEM((CH, D), f32),
            pltpu.VMEM((2, CH), i32),
            pltpu.VMEM((2, CH), i32),
            pltpu.VMEM((2, CH), i32),
            pltpu.VMEM((2, CH), i32),
            pltpu.SemaphoreType.DMA,
            pltpu.SemaphoreType.DMA,
            pltpu.SemaphoreType.DMA,
            pltpu.SemaphoreType.DMA,
            pltpu.SemaphoreType.DMA,
            pltpu.SemaphoreType.DMA,
            pltpu.SemaphoreType.DMA,
            pltpu.SemaphoreType.DMA,
            pltpu.VMEM_SHARED((N_PAD, D), f32),
        ],
        compiler_params=pltpu.CompilerParams(needs_layout_passes=False,
                                             use_tc_tiling_on_sc=False),
    )(kk, qq, vv, er, idx, jnp.zeros((RPT, D), f32))


def _k5a_body(out2_ref, skip_ref, rbias_ref, h2_ref, stats_ref):
    i = pl.program_id(0)
    h2 = out2_ref[0] + out2_ref[1] + skip_ref[...] + rbias_ref[...][None, :]
    h2 = jnp.where(h2 > 0, h2, 0.01 * h2)
    h2_ref[...] = h2

    @pl.when(i == 0)
    def _():
        stats_ref[...] = jnp.zeros_like(stats_ref)

    ssum = jnp.sum(h2, axis=0, keepdims=True)
    ssq = jnp.sum(h2 * h2, axis=0, keepdims=True)
    stats_ref[...] += jnp.concatenate(
        [ssum, ssq, jnp.zeros((6, D), f32)], axis=0)


def _k5a(out2, skip, rbias):
    return pl.pallas_call(
        _k5a_body,
        grid=(N // BN,),
        in_specs=[
            pl.BlockSpec((NC, BN, D), lambda i: (0, i, 0)),
            pl.BlockSpec((BN, D), lambda i: (i, 0)),
            pl.BlockSpec((D,), lambda i: (0,)),
        ],
        out_specs=[
            pl.BlockSpec((BN, D), lambda i: (i, 0)),
            pl.BlockSpec((8, D), lambda i: (0, 0)),
        ],
        out_shape=[
            jax.ShapeDtypeStruct((N, D), f32),
            jax.ShapeDtypeStruct((8, D), f32),
        ],
    )(out2, skip, rbias)


def _k5b_body(h2_ref, stats_ref, gamma_ref, beta_ref, out_ref):
    mu = stats_ref[0, :] * (1.0 / N)
    var = stats_ref[1, :] * (1.0 / N) - mu * mu
    inv = 1.0 / jnp.sqrt(var + 1e-5)
    out_ref[...] = ((h2_ref[...] - mu[None, :]) * (inv * gamma_ref[...])[None, :]
                    + beta_ref[...][None, :])


def _k5b(h2, stats, gamma, beta):
    return pl.pallas_call(
        _k5b_body,
        grid=(N // BN,),
        in_specs=[
            pl.BlockSpec((BN, D), lambda i: (i, 0)),
            pl.BlockSpec((8, D), lambda i: (0, 0)),
            pl.BlockSpec((D,), lambda i: (0,)),
            pl.BlockSpec((D,), lambda i: (0,)),
        ],
        out_specs=pl.BlockSpec((BN, D), lambda i: (i, 0)),
        out_shape=jax.ShapeDtypeStruct((N, D), f32),
    )(h2, stats, gamma, beta)


@jax.jit
def kernel(x, edge_index, edge_attr, gat_Wl, gat_bl, gat_Wr, gat_br, gat_att,
           gat_We, gat_bias, res_Wk, res_bk, res_Wq, res_bq, res_Wv, res_bv,
           res_We, res_Wskip, res_bias, bn_gamma, bn_beta):
    idx = jnp.stack([edge_index[0].astype(i32).reshape(NW, NCHUNK, CH),
                     edge_index[1].astype(i32).reshape(NW, NCHUNK, CH)],
                    axis=2)

    eg, er, easum = _eprep(edge_attr, gat_We, res_We)
    xl, xr, lout, lden = _nprep(easum, x, gat_Wl, gat_bl, gat_Wr, gat_br,
                                gat_att, gat_We)
    outsc, densc = _gat_edges(xl, xr, eg, idx, gat_att)
    kk, qq, vv, skip = _combine(outsc, densc, lout, lden, gat_bias,
                                res_Wk, res_bk, res_Wq, res_bq,
                                res_Wv, res_bv, res_Wskip)
    out2 = _res_edges(kk, qq, vv, er, idx)
    h2, stats = _k5a(out2, skip, res_bias)
    return _k5b(h2, stats, bn_gamma, bn_beta)
